# baseline (speedup 1.0000x reference)
import jax
import jax.numpy as jnp
from jax import lax
from jax.experimental import pallas as pl
from jax.experimental.pallas import tpu as pltpu

_T_TILE = 64
_U = 8


def _lstm_tile_kernel(x_ref, whh_ref, wxb_ref, out_ref, hn_ref, cn_ref):
    Bb, H = hn_ref.shape
    H3 = 3 * H
    half = Bb // 2
    tid = pl.program_id(1)

    @pl.when(tid == 0)
    def _init():
        hn_ref[...] = jnp.zeros_like(hn_ref)
        cn_ref[...] = jnp.zeros_like(cn_ref)

    whh = whh_ref[...]
    wih = wxb_ref[0:1, :]
    bias = wxb_ref[1:2, :]

    def cell(x_col, h, c):
        gates = (jnp.dot(h.astype(jnp.bfloat16), whh,
                         preferred_element_type=jnp.float32)
                 + x_col * wih + bias)
        sig = 0.5 * jnp.tanh(0.5 * gates[:, :H3]) + 0.5
        g_gate = jnp.tanh(gates[:, H3:])
        c = sig[:, H:2 * H] * c + sig[:, :H] * g_gate
        h = sig[:, 2 * H:H3] * jnp.tanh(c)
        return h, c

    def chunk_body(ci, carry):
        h0, c0, h1, c1 = carry
        base = pl.multiple_of(ci * _U, _U)
        xs = x_ref[pl.ds(base, _U), :, :]
        off = pl.multiple_of(ci * (_U * H), _U * H)
        for j in range(_U):
            h0, c0 = cell(xs[j, :half, :], h0, c0)
            h1, c1 = cell(xs[j, half:, :], h1, c1)
            out_ref[0:half, pl.ds(off + j * H, H)] = h0
            out_ref[half:Bb, pl.ds(off + j * H, H)] = h1
        return h0, c0, h1, c1

    carry = (hn_ref[0:half, :], cn_ref[0:half, :],
             hn_ref[half:Bb, :], cn_ref[half:Bb, :])
    h0, c0, h1, c1 = lax.fori_loop(0, _T_TILE // _U, chunk_body, carry)

    hn_ref[0:half, :] = h0
    hn_ref[half:Bb, :] = h1
    cn_ref[0:half, :] = c0
    cn_ref[half:Bb, :] = c1


def kernel(x, w_ih, w_hh, b_ih, b_hh):
    B, T = x.shape
    H = w_hh.shape[1]

    def perm_gates(a, axis):
        i, f, g, o = jnp.split(a.astype(jnp.float32), 4, axis=axis)
        return jnp.concatenate([i, f, o, g], axis=axis)

    whh_t = perm_gates(jnp.transpose(w_hh), axis=1).astype(jnp.bfloat16)
    wih_row = perm_gates(w_ih.reshape(1, 4 * H), axis=1)
    bias = perm_gates((b_ih + b_hh).reshape(1, 4 * H), axis=1)
    wxb = jnp.concatenate([wih_row, bias], axis=0)

    x_tb1 = jnp.transpose(x.astype(jnp.float32))[:, :, None]

    t_tile = _T_TILE
    num_tiles = T // t_tile
    b_block = B // 2 if (B % 32 == 0) else B
    num_b = B // b_block

    out_flat, h_n, c_n = pl.pallas_call(
        _lstm_tile_kernel,
        grid=(num_b, num_tiles),
        in_specs=[
            pl.BlockSpec((t_tile, b_block, 1), lambda b, t: (t, b, 0)),
            pl.BlockSpec((H, 4 * H), lambda b, t: (0, 0)),
            pl.BlockSpec((2, 4 * H), lambda b, t: (0, 0)),
        ],
        out_specs=(
            pl.BlockSpec((b_block, t_tile * H), lambda b, t: (b, t)),
            pl.BlockSpec((b_block, H), lambda b, t: (b, 0)),
            pl.BlockSpec((b_block, H), lambda b, t: (b, 0)),
        ),
        out_shape=(
            jax.ShapeDtypeStruct((B, T * H), jnp.float32),
            jax.ShapeDtypeStruct((B, H), jnp.float32),
            jax.ShapeDtypeStruct((B, H), jnp.float32),
        ),
        compiler_params=pltpu.CompilerParams(
            dimension_semantics=("parallel", "arbitrary")),
    )(x_tb1, whh_t, wxb)

    output = out_flat.reshape(B, T, H)
    return output, (h_n[None, ...], c_n[None, ...])


# speedup vs baseline: 1.8028x; 1.8028x over previous
import jax
import jax.numpy as jnp
from jax.experimental import pallas as pl
from jax.experimental.pallas import tpu as pltpu

_T_TILE = 128
_U = 8


def _lstm_tile_kernel(x_ref, whh_ref, wxb_ref, out_ref, hn_ref, cn_ref,
                      scr_ref):
    Bb, H = hn_ref.shape
    H3 = 3 * H
    half = Bb // 2
    tid = pl.program_id(1)

    @pl.when(tid == 0)
    def _init():
        hn_ref[...] = jnp.zeros_like(hn_ref)
        cn_ref[...] = jnp.zeros_like(cn_ref)

    whh = whh_ref[...]
    wih = wxb_ref[0:1, :]
    bias = wxb_ref[1:2, :]
    xb = x_ref[...]

    def cell(x_col, h, c):
        gates = (jnp.dot(h.astype(jnp.bfloat16), whh,
                         preferred_element_type=jnp.float32)
                 + x_col * wih + bias)
        sig = 0.5 * jnp.tanh(0.5 * gates[:, :H3]) + 0.5
        g_gate = jnp.tanh(gates[:, H3:])
        c = sig[:, H:2 * H] * c + sig[:, :H] * g_gate
        h = sig[:, 2 * H:H3] * jnp.tanh(c)
        return h, c

    h0 = hn_ref[0:half, :]
    h1 = hn_ref[half:Bb, :]
    c0 = cn_ref[0:half, :]
    c1 = cn_ref[half:Bb, :]

    for ci in range(_T_TILE // _U):
        for j in range(_U):
            t = ci * _U + j
            h0, c0 = cell(xb[0:half, t:t + 1], h0, c0)
            h1, c1 = cell(xb[half:Bb, t:t + 1], h1, c1)
            scr_ref[0:half, j * H:(j + 1) * H] = h0
            scr_ref[half:Bb, j * H:(j + 1) * H] = h1
        rg = min(32, Bb)
        for r in range(0, Bb, rg):
            out_ref[r:r + rg, ci * _U:(ci + 1) * _U, :] = (
                scr_ref[r:r + rg, :].reshape(rg, _U, H))

    hn_ref[0:half, :] = h0
    hn_ref[half:Bb, :] = h1
    cn_ref[0:half, :] = c0
    cn_ref[half:Bb, :] = c1


def kernel(x, w_ih, w_hh, b_ih, b_hh):
    B, T = x.shape
    H = w_hh.shape[1]

    def perm_gates(a, axis):
        i, f, g, o = jnp.split(a.astype(jnp.float32), 4, axis=axis)
        return jnp.concatenate([i, f, o, g], axis=axis)

    whh_t = perm_gates(jnp.transpose(w_hh), axis=1).astype(jnp.bfloat16)
    wih_row = perm_gates(w_ih.reshape(1, 4 * H), axis=1)
    bias = perm_gates((b_ih + b_hh).reshape(1, 4 * H), axis=1)
    wxb = jnp.concatenate([wih_row, bias], axis=0)

    t_tile = _T_TILE
    num_tiles = T // t_tile
    b_block = B // 2 if (B % 32 == 0) else B
    num_b = B // b_block

    out, h_n, c_n = pl.pallas_call(
        _lstm_tile_kernel,
        grid=(num_b, num_tiles),
        in_specs=[
            pl.BlockSpec((b_block, t_tile), lambda b, t: (b, t)),
            pl.BlockSpec((H, 4 * H), lambda b, t: (0, 0)),
            pl.BlockSpec((2, 4 * H), lambda b, t: (0, 0)),
        ],
        out_specs=(
            pl.BlockSpec((b_block, t_tile, H), lambda b, t: (b, t, 0)),
            pl.BlockSpec((b_block, H), lambda b, t: (b, 0)),
            pl.BlockSpec((b_block, H), lambda b, t: (b, 0)),
        ),
        out_shape=(
            jax.ShapeDtypeStruct((B, T, H), jnp.float32),
            jax.ShapeDtypeStruct((B, H), jnp.float32),
            jax.ShapeDtypeStruct((B, H), jnp.float32),
        ),
        scratch_shapes=[pltpu.VMEM((b_block, _U * H), jnp.float32)],
        compiler_params=pltpu.CompilerParams(
            dimension_semantics=("parallel", "arbitrary")),
    )(x.astype(jnp.float32), whh_t, wxb)

    return out, (h_n[None, ...], c_n[None, ...])


# speedup vs baseline: 2.1937x; 1.2169x over previous
import jax
import jax.numpy as jnp
from jax.experimental import pallas as pl
from jax.experimental.pallas import tpu as pltpu

_T_TILE = 128
_U = 8


def _lstm_tile_kernel(x_ref, waug_ref, out_ref, hn_ref, cn_ref, scr_ref):
    Bb, H = hn_ref.shape
    H3 = 3 * H
    half = Bb // 2
    tid = pl.program_id(1)

    @pl.when(tid == 0)
    def _init():
        hn_ref[...] = jnp.zeros_like(hn_ref)
        cn_ref[...] = jnp.zeros_like(cn_ref)

    waug = waug_ref[...]
    xb = x_ref[...].astype(jnp.bfloat16)
    ones_col = jnp.ones((half, 1), jnp.bfloat16)

    def cell(x_col, h, c):
        aug = jnp.concatenate([h, x_col, ones_col], axis=1)
        gates = jnp.dot(aug, waug, preferred_element_type=jnp.float32)
        sig = jnp.tanh(gates[:, :H3]) * 0.5 + 0.5
        g_gate = jnp.tanh(gates[:, H3:])
        c = sig[:, H:2 * H] * c + sig[:, :H] * g_gate
        h32 = sig[:, 2 * H:H3] * jnp.tanh(c)
        return h32, h32.astype(jnp.bfloat16), c

    h0 = hn_ref[0:half, :].astype(jnp.bfloat16)
    h1 = hn_ref[half:Bb, :].astype(jnp.bfloat16)
    c0 = cn_ref[0:half, :]
    c1 = cn_ref[half:Bb, :]

    for ci in range(_T_TILE // _U):
        for j in range(_U):
            t = ci * _U + j
            h0_32, h0, c0 = cell(xb[0:half, t:t + 1], h0, c0)
            h1_32, h1, c1 = cell(xb[half:Bb, t:t + 1], h1, c1)
            scr_ref[0:half, j * H:(j + 1) * H] = h0_32
            scr_ref[half:Bb, j * H:(j + 1) * H] = h1_32
        rg = min(32, Bb)
        for r in range(0, Bb, rg):
            out_ref[r:r + rg, ci * _U:(ci + 1) * _U, :] = (
                scr_ref[r:r + rg, :].reshape(rg, _U, H))

    hn_ref[0:half, :] = h0_32
    hn_ref[half:Bb, :] = h1_32
    cn_ref[0:half, :] = c0
    cn_ref[half:Bb, :] = c1


def kernel(x, w_ih, w_hh, b_ih, b_hh):
    B, T = x.shape
    H = w_hh.shape[1]

    def perm_gates(a, axis):
        i, f, g, o = jnp.split(a.astype(jnp.float32), 4, axis=axis)
        return jnp.concatenate([i, f, o, g], axis=axis)

    whh_t = perm_gates(jnp.transpose(w_hh), axis=1)
    wih_row = perm_gates(w_ih.reshape(1, 4 * H), axis=1)
    bias = perm_gates((b_ih + b_hh).reshape(1, 4 * H), axis=1)
    waug = jnp.concatenate([whh_t, wih_row, bias], axis=0)
    col_scale = jnp.concatenate([jnp.full((1, 3 * H), 0.5, jnp.float32),
                                 jnp.ones((1, H), jnp.float32)], axis=1)
    waug = (waug * col_scale).astype(jnp.bfloat16)

    t_tile = _T_TILE
    num_tiles = T // t_tile
    b_block = B // 2 if (B % 32 == 0) else B
    num_b = B // b_block

    out, h_n, c_n = pl.pallas_call(
        _lstm_tile_kernel,
        grid=(num_b, num_tiles),
        in_specs=[
            pl.BlockSpec((b_block, t_tile), lambda b, t: (b, t)),
            pl.BlockSpec((H + 2, 4 * H), lambda b, t: (0, 0)),
        ],
        out_specs=(
            pl.BlockSpec((b_block, t_tile, H), lambda b, t: (b, t, 0)),
            pl.BlockSpec((b_block, H), lambda b, t: (b, 0)),
            pl.BlockSpec((b_block, H), lambda b, t: (b, 0)),
        ),
        out_shape=(
            jax.ShapeDtypeStruct((B, T, H), jnp.float32),
            jax.ShapeDtypeStruct((B, H), jnp.float32),
            jax.ShapeDtypeStruct((B, H), jnp.float32),
        ),
        scratch_shapes=[pltpu.VMEM((b_block, _U * H), jnp.float32)],
        compiler_params=pltpu.CompilerParams(
            dimension_semantics=("parallel", "arbitrary")),
    )(x.astype(jnp.float32), waug)

    return out, (h_n[None, ...], c_n[None, ...])


# speedup vs baseline: 2.2240x; 1.0138x over previous
import jax
import jax.numpy as jnp
from jax.experimental import pallas as pl
from jax.experimental.pallas import tpu as pltpu

_T_TILE = 128
_U = 8


def _lstm_tile_kernel(x_ref, waug_ref, out_ref, hn_ref, cn_ref, scr_ref):
    Bb, H = hn_ref.shape
    H3 = 3 * H
    half = Bb // 2
    tid = pl.program_id(1)

    @pl.when(tid == 0)
    def _init():
        hn_ref[...] = jnp.zeros_like(hn_ref)
        cn_ref[...] = jnp.zeros_like(cn_ref)

    ns = 4
    sr = Bb // ns
    waug = waug_ref[...]
    xb = x_ref[...].astype(jnp.bfloat16)
    ones_col = jnp.ones((sr, 1), jnp.bfloat16)

    def cell(x_col, h, c):
        aug = jnp.concatenate([h, x_col, ones_col], axis=1)
        gates = jnp.dot(aug, waug, preferred_element_type=jnp.float32)
        sig = jnp.tanh(gates[:, :H3]) * 0.5 + 0.5
        g_gate = jnp.tanh(gates[:, H3:])
        c = sig[:, H:2 * H] * c + sig[:, :H] * g_gate
        h32 = sig[:, 2 * H:H3] * jnp.tanh(c)
        return h32, h32.astype(jnp.bfloat16), c

    hs = [hn_ref[k * sr:(k + 1) * sr, :].astype(jnp.bfloat16)
          for k in range(ns)]
    cs = [cn_ref[k * sr:(k + 1) * sr, :] for k in range(ns)]
    hs32 = [None] * ns

    for ci in range(_T_TILE // _U):
        for j in range(_U):
            t = ci * _U + j
            for k in range(ns):
                hs32[k], hs[k], cs[k] = cell(
                    xb[k * sr:(k + 1) * sr, t:t + 1], hs[k], cs[k])
                scr_ref[k * sr:(k + 1) * sr, j * H:(j + 1) * H] = hs32[k]
        rg = min(32, Bb)
        for r in range(0, Bb, rg):
            out_ref[r:r + rg, ci * _U:(ci + 1) * _U, :] = (
                scr_ref[r:r + rg, :].reshape(rg, _U, H))

    for k in range(ns):
        hn_ref[k * sr:(k + 1) * sr, :] = hs32[k]
        cn_ref[k * sr:(k + 1) * sr, :] = cs[k]


def kernel(x, w_ih, w_hh, b_ih, b_hh):
    B, T = x.shape
    H = w_hh.shape[1]

    def perm_gates(a, axis):
        i, f, g, o = jnp.split(a.astype(jnp.float32), 4, axis=axis)
        return jnp.concatenate([i, f, o, g], axis=axis)

    whh_t = perm_gates(jnp.transpose(w_hh), axis=1)
    wih_row = perm_gates(w_ih.reshape(1, 4 * H), axis=1)
    bias = perm_gates((b_ih + b_hh).reshape(1, 4 * H), axis=1)
    waug = jnp.concatenate([whh_t, wih_row, bias], axis=0)
    col_scale = jnp.concatenate([jnp.full((1, 3 * H), 0.5, jnp.float32),
                                 jnp.ones((1, H), jnp.float32)], axis=1)
    waug = (waug * col_scale).astype(jnp.bfloat16)

    t_tile = _T_TILE
    num_tiles = T // t_tile
    b_block = B // 2 if (B % 32 == 0) else B
    num_b = B // b_block

    out, h_n, c_n = pl.pallas_call(
        _lstm_tile_kernel,
        grid=(num_b, num_tiles),
        in_specs=[
            pl.BlockSpec((b_block, t_tile), lambda b, t: (b, t)),
            pl.BlockSpec((H + 2, 4 * H), lambda b, t: (0, 0)),
        ],
        out_specs=(
            pl.BlockSpec((b_block, t_tile, H), lambda b, t: (b, t, 0)),
            pl.BlockSpec((b_block, H), lambda b, t: (b, 0)),
            pl.BlockSpec((b_block, H), lambda b, t: (b, 0)),
        ),
        out_shape=(
            jax.ShapeDtypeStruct((B, T, H), jnp.float32),
            jax.ShapeDtypeStruct((B, H), jnp.float32),
            jax.ShapeDtypeStruct((B, H), jnp.float32),
        ),
        scratch_shapes=[pltpu.VMEM((b_block, _U * H), jnp.float32)],
        compiler_params=pltpu.CompilerParams(
            dimension_semantics=("parallel", "arbitrary")),
    )(x.astype(jnp.float32), waug)

    return out, (h_n[None, ...], c_n[None, ...])
